# baseline (speedup 1.0000x reference)
import functools

import jax
import jax.numpy as jnp
from jax import lax
from jax.experimental import pallas as pl
from jax.experimental.pallas import tpu as pltpu
from jax.experimental.pallas import tpu_sc as plsc

BATCH = 16384
EMB = 32
NC = 2
NS = 16
NW = NC * NS
BPW = BATCH // NW
IDX_CHUNK = 128
NCH = BPW // IDX_CHUNK

_mesh = plsc.VectorSubcoreMesh(core_axis_name="c", subcore_axis_name="s")


@functools.partial(
    pl.kernel,
    out_type=jax.ShapeDtypeStruct((BATCH,), jnp.float32),
    mesh=_mesh,
    scratch_types=[
        pltpu.VMEM((NCH, IDX_CHUNK), jnp.int32),
        pltpu.VMEM((NCH, IDX_CHUNK), jnp.int32),
        pltpu.VMEM((BPW, EMB), jnp.float32),
        pltpu.VMEM((BPW, EMB), jnp.float32),
        pltpu.VMEM((BPW,), jnp.float32),
        pltpu.SemaphoreType.DMA,
        pltpu.SemaphoreType.DMA,
    ],
    compiler_params=pltpu.CompilerParams(
        needs_layout_passes=False, use_tc_tiling_on_sc=False),
)
def _mf_sc(u_hbm, v_hbm, ue_hbm, ve_hbm, out_hbm,
           uidx, vidx, ue_v, ve_v, out_v, sem_u, sem_i):
    wid = lax.axis_index("s") * NC + lax.axis_index("c")
    base = wid * NCH
    pltpu.sync_copy(u_hbm.at[pl.ds(base, NCH)], uidx)
    pltpu.sync_copy(v_hbm.at[pl.ds(base, NCH)], vidx)
    copies = []
    for c in range(NCH):
        rows = pl.ds(c * IDX_CHUNK, IDX_CHUNK)
        copies.append(pltpu.async_copy(ue_hbm.at[uidx.at[c]], ue_v.at[rows], sem_u))
        copies.append(pltpu.async_copy(ve_hbm.at[vidx.at[c]], ve_v.at[rows], sem_i))
    for cp in copies:
        cp.wait()

    lanes = lax.iota(jnp.int32, 16)

    def body(g, carry):
        rows = g * 16 + lanes
        acc = jnp.zeros((16,), jnp.float32)
        for j in range(EMB):
            col = jnp.full((16,), j, jnp.int32)
            a = plsc.load_gather(ue_v, [rows, col])
            b = plsc.load_gather(ve_v, [rows, col])
            acc = acc + a * b
        out_v[pl.ds(g * 16, 16)] = acc
        return carry

    lax.fori_loop(0, BPW // 16, body, 0)
    pltpu.sync_copy(out_v, out_hbm.at[pl.ds(wid * BPW, BPW)])


def kernel(u, v, user_emb, item_emb):
    u2 = u.astype(jnp.int32).reshape(NW * NCH, IDX_CHUNK)
    v2 = v.astype(jnp.int32).reshape(NW * NCH, IDX_CHUNK)
    return _mf_sc(u2, v2, user_emb, item_emb)
